# baseline (device time: 114789 ns/iter reference)
import jax
import jax.numpy as jnp
from jax import lax
from jax.experimental import pallas as pl
from jax.experimental.pallas import tpu as pltpu

N_DEV = 8
B = 4
Sq = 256
Hq = 8
Hkv = 2
Dh = 128
Dm = 1024
C = 1024
G = Hq // Hkv
R = G * Sq
SCALE = 0.08838834764831843
HOPS = 3

_MESH = pl.DeviceIdType.MESH


def _gray(v):
    return jnp.where(v < 4, v, 11 - v)


def kernel(x, Wq, Wo, K_ext, V_ext):
    def body(x_ref, wq_ref, wo_ref, k_ref, v_ref, out_ref,
             rbuf, lbuf, mbuf, rl, ll, mlb, acc_run_ref,
             r_send, r_recv, l_send, l_recv,
             rl_send, rl_recv, ll_send, ll_recv,
             m_send, m_recv, ml2_send, ml2_recv):
        my = lax.axis_index("i")
        vi = _gray(my)
        right = _gray(lax.rem(vi + 1, N_DEV))
        left = _gray(lax.rem(vi + N_DEV - 1, N_DEV))
        is_even = lax.rem(vi, 2) == 0
        partner_v = lax.rem(vi + jnp.where(is_even, 3, 5), N_DEV)
        partner = _gray(partner_v)

        barrier = pltpu.get_barrier_semaphore()
        for nbr in (left, right, partner):
            pl.semaphore_signal(barrier, inc=1, device_id=(nbr,),
                                device_id_type=_MESH)
        pl.semaphore_wait(barrier, 3)

        xqT = lax.dot_general(
            wq_ref[:].astype(jnp.bfloat16),
            x_ref[:].reshape(B * Sq, Dm).astype(jnp.bfloat16),
            (((0,), (1,)), ((), ())),
            preferred_element_type=jnp.float32) * SCALE
        xq16 = xqT.astype(jnp.bfloat16)

        l_run = [[None] * Hkv for _ in range(B)]
        for b in range(B):
            for g in range(Hkv):
                qT = jnp.concatenate(
                    [xq16[(g * G + qi) * Dh:(g * G + qi + 1) * Dh,
                          b * Sq:(b + 1) * Sq]
                     for qi in range(G)], axis=1)
                kg = k_ref[b, :, g, :].astype(jnp.bfloat16)
                sT = lax.dot_general(
                    kg, qT, (((1,), (0,)), ((), ())),
                    preferred_element_type=jnp.float32)
                pT = jnp.exp(sT).astype(jnp.bfloat16)
                l_b = jnp.sum(pT.astype(jnp.float32), axis=0,
                              keepdims=True)
                vg = v_ref[b, :, g, :].astype(jnp.bfloat16)
                accT = lax.dot_general(
                    vg, pT, (((0,), (0,)), ((), ())),
                    preferred_element_type=jnp.float32)
                acc_run_ref[b, g, :, :] = accT
                acc16 = accT.astype(jnp.bfloat16)
                l16 = l_b.astype(jnp.bfloat16)
                rbuf[0, b, g, :, :] = acc16
                lbuf[0, b, g, :, :] = acc16
                rl[0, b, g, :, :] = l16
                ll[0, b, g, :, :] = l16
                l_run[b][g] = l_b

        def make(buf, sems_s, sems_r, ss, rs, dst):
            return pltpu.make_async_remote_copy(
                src_ref=buf.at[ss], dst_ref=buf.at[rs],
                send_sem=sems_s.at[ss], recv_sem=sems_r.at[rs],
                device_id=(dst,), device_id_type=_MESH)

        def make_match(src):
            return pltpu.make_async_remote_copy(
                src_ref=src, dst_ref=mbuf,
                send_sem=m_send, recv_sem=m_recv,
                device_id=(partner,), device_id_type=_MESH)

        def make_match_l(src):
            return pltpu.make_async_remote_copy(
                src_ref=src, dst_ref=mlb,
                send_sem=ml2_send, recv_sem=ml2_recv,
                device_id=(partner,), device_id_type=_MESH)

        def start_ring(h):
            ss, rs = h % 2, (h + 1) % 2
            rdmas = [make(rbuf, r_send, r_recv, ss, rs, right),
                     make(rl, rl_send, rl_recv, ss, rs, right),
                     make(lbuf, l_send, l_recv, ss, rs, left),
                     make(ll, ll_send, ll_recv, ss, rs, left)]
            for r in rdmas:
                r.start()
            return rdmas

        def merge(rs, with_match):
            for b in range(B):
                for g in range(Hkv):
                    inc = (rbuf[rs, b, g, :, :].astype(jnp.float32)
                           + lbuf[rs, b, g, :, :].astype(jnp.float32))
                    l_inc = (rl[rs, b, g, :, :].astype(jnp.float32)
                             + ll[rs, b, g, :, :].astype(jnp.float32))
                    if with_match:
                        inc = inc + mbuf[b, g, :, :].astype(jnp.float32)
                        l_inc = l_inc + mlb[b, g, :, :].astype(jnp.float32)
                    acc_run_ref[b, g, :, :] = acc_run_ref[b, g, :, :] + inc
                    l_run[b][g] = l_run[b][g] + l_inc

        inflight = start_ring(0)
        for r in inflight:
            r.wait()

        inflight = start_ring(1)

        @pl.when(is_even)
        def _():
            make_match(rbuf.at[1]).start()
            make_match_l(rl.at[1]).start()

        @pl.when(jnp.logical_not(is_even))
        def _():
            make_match(lbuf.at[1]).start()
            make_match_l(ll.at[1]).start()

        merge(1, with_match=False)
        for r in inflight:
            r.wait()
        make_match(rbuf.at[1]).wait()
        make_match_l(rl.at[1]).wait()

        inflight = start_ring(2)
        merge(0, with_match=True)
        for r in inflight:
            r.wait()
        for b in range(B):
            for g in range(Hkv):
                l_run[b][g] = (l_run[b][g]
                               + rl[1, b, g, :, :].astype(jnp.float32)
                               + ll[1, b, g, :, :].astype(jnp.float32))

        wo16 = wo_ref[:].astype(jnp.bfloat16)
        for b in range(B):
            total = None
            for g in range(Hkv):
                inv_l = 1.0 / l_run[b][g]
                for qi in range(G):
                    h_idx = g * G + qi
                    sl = slice(qi * Sq, (qi + 1) * Sq)
                    acc_full = (acc_run_ref[b, g, :, sl]
                                + rbuf[1, b, g, :, sl].astype(jnp.float32)
                                + lbuf[1, b, g, :, sl].astype(jnp.float32))
                    oT = (acc_full * inv_l[:, sl]).astype(jnp.bfloat16)
                    contrib = lax.dot_general(
                        oT, wo16[h_idx * Dh:(h_idx + 1) * Dh, :],
                        (((0,), (0,)), ((), ())),
                        preferred_element_type=jnp.float32)
                    total = contrib if total is None else total + contrib
            out_ref[b, :, :] = total

    return pl.pallas_call(
        body,
        out_shape=jax.ShapeDtypeStruct((B, Sq, Dm), jnp.float32),
        in_specs=[pl.BlockSpec(memory_space=pltpu.VMEM)] * 5,
        out_specs=pl.BlockSpec(memory_space=pltpu.VMEM),
        scratch_shapes=[
            pltpu.VMEM((2, B, Hkv, Dh, R), jnp.bfloat16),
            pltpu.VMEM((2, B, Hkv, Dh, R), jnp.bfloat16),
            pltpu.VMEM((B, Hkv, Dh, R), jnp.bfloat16),
            pltpu.VMEM((2, B, Hkv, 1, R), jnp.bfloat16),
            pltpu.VMEM((2, B, Hkv, 1, R), jnp.bfloat16),
            pltpu.VMEM((B, Hkv, 1, R), jnp.bfloat16),
            pltpu.VMEM((B, Hkv, Dh, R), jnp.float32),
            pltpu.SemaphoreType.DMA((2,)),
            pltpu.SemaphoreType.DMA((2,)),
            pltpu.SemaphoreType.DMA((2,)),
            pltpu.SemaphoreType.DMA((2,)),
            pltpu.SemaphoreType.DMA((2,)),
            pltpu.SemaphoreType.DMA((2,)),
            pltpu.SemaphoreType.DMA((2,)),
            pltpu.SemaphoreType.DMA((2,)),
            pltpu.SemaphoreType.DMA,
            pltpu.SemaphoreType.DMA,
            pltpu.SemaphoreType.DMA,
            pltpu.SemaphoreType.DMA,
        ],
        compiler_params=pltpu.CompilerParams(
            collective_id=0, vmem_limit_bytes=100 * 1024 * 1024),
    )(x, Wq, Wo, K_ext, V_ext)


# device time: 113540 ns/iter; 1.0110x vs baseline; 1.0110x over previous
import jax
import jax.numpy as jnp
from jax import lax
from jax.experimental import pallas as pl
from jax.experimental.pallas import tpu as pltpu

N_DEV = 8
B = 4
Sq = 256
Hq = 8
Hkv = 2
Dh = 128
Dm = 1024
C = 1024
G = Hq // Hkv
R = G * Sq
SCALE = 0.08838834764831843
HOPS = 3

_MESH = pl.DeviceIdType.MESH


def _gray(v):
    return jnp.where(v < 4, v, 11 - v)


def kernel(x, Wq, Wo, K_ext, V_ext):
    def body(x_ref, wq_ref, wo_ref, k_ref, v_ref, out_ref,
             rbuf, lbuf, mbuf, rl, ll, mlb, acc_run_ref,
             r_send, r_recv, l_send, l_recv,
             rl_send, rl_recv, ll_send, ll_recv,
             m_send, m_recv, ml2_send, ml2_recv):
        my = lax.axis_index("i")
        vi = _gray(my)
        right = _gray(lax.rem(vi + 1, N_DEV))
        left = _gray(lax.rem(vi + N_DEV - 1, N_DEV))
        is_even = lax.rem(vi, 2) == 0
        partner_v = lax.rem(vi + jnp.where(is_even, 3, 5), N_DEV)
        partner = _gray(partner_v)

        barrier = pltpu.get_barrier_semaphore()
        for nbr in (left, right, partner):
            pl.semaphore_signal(barrier, inc=1, device_id=(nbr,),
                                device_id_type=_MESH)
        pl.semaphore_wait(barrier, 3)

        xqT = lax.dot_general(
            wq_ref[:].astype(jnp.bfloat16),
            x_ref[:].reshape(B * Sq, Dm).astype(jnp.bfloat16),
            (((0,), (1,)), ((), ())),
            preferred_element_type=jnp.float32) * SCALE
        xq16 = xqT.astype(jnp.bfloat16)

        l_run = [[None] * Hkv for _ in range(B)]
        for b in range(B):
            for g in range(Hkv):
                qT = jnp.concatenate(
                    [xq16[(g * G + qi) * Dh:(g * G + qi + 1) * Dh,
                          b * Sq:(b + 1) * Sq]
                     for qi in range(G)], axis=1)
                kg = k_ref[b, :, g, :].astype(jnp.bfloat16)
                sT = lax.dot_general(
                    kg, qT, (((1,), (0,)), ((), ())),
                    preferred_element_type=jnp.float32)
                pT = jnp.exp(sT)
                l_b = jnp.sum(pT, axis=0, keepdims=True)
                vg = v_ref[b, :, g, :].astype(jnp.bfloat16)
                accT = lax.dot_general(
                    vg, pT.astype(jnp.bfloat16), (((0,), (0,)), ((), ())),
                    preferred_element_type=jnp.float32)
                acc_run_ref[b, g, :, :] = accT
                acc16 = accT.astype(jnp.bfloat16)
                l16 = l_b.astype(jnp.bfloat16)
                rbuf[0, b, g, :, :] = acc16
                lbuf[0, b, g, :, :] = acc16
                rl[0, b, g, :, :] = l16
                ll[0, b, g, :, :] = l16
                l_run[b][g] = l_b

        def make(buf, sems_s, sems_r, ss, rs, dst):
            return pltpu.make_async_remote_copy(
                src_ref=buf.at[ss], dst_ref=buf.at[rs],
                send_sem=sems_s.at[ss], recv_sem=sems_r.at[rs],
                device_id=(dst,), device_id_type=_MESH)

        def make_match(src):
            return pltpu.make_async_remote_copy(
                src_ref=src, dst_ref=mbuf,
                send_sem=m_send, recv_sem=m_recv,
                device_id=(partner,), device_id_type=_MESH)

        def make_match_l(src):
            return pltpu.make_async_remote_copy(
                src_ref=src, dst_ref=mlb,
                send_sem=ml2_send, recv_sem=ml2_recv,
                device_id=(partner,), device_id_type=_MESH)

        def start_ring(h):
            ss, rs = h % 2, (h + 1) % 2
            rdmas = [make(rbuf, r_send, r_recv, ss, rs, right),
                     make(rl, rl_send, rl_recv, ss, rs, right),
                     make(lbuf, l_send, l_recv, ss, rs, left),
                     make(ll, ll_send, ll_recv, ss, rs, left)]
            for r in rdmas:
                r.start()
            return rdmas

        def merge(rs, with_match):
            for b in range(B):
                for g in range(Hkv):
                    inc = (rbuf[rs, b, g, :, :].astype(jnp.float32)
                           + lbuf[rs, b, g, :, :].astype(jnp.float32))
                    l_inc = (rl[rs, b, g, :, :].astype(jnp.float32)
                             + ll[rs, b, g, :, :].astype(jnp.float32))
                    if with_match:
                        inc = inc + mbuf[b, g, :, :].astype(jnp.float32)
                        l_inc = l_inc + mlb[b, g, :, :].astype(jnp.float32)
                    acc_run_ref[b, g, :, :] = acc_run_ref[b, g, :, :] + inc
                    l_run[b][g] = l_run[b][g] + l_inc

        inflight = start_ring(0)
        for r in inflight:
            r.wait()

        inflight = start_ring(1)

        @pl.when(is_even)
        def _():
            make_match(rbuf.at[1]).start()
            make_match_l(rl.at[1]).start()

        @pl.when(jnp.logical_not(is_even))
        def _():
            make_match(lbuf.at[1]).start()
            make_match_l(ll.at[1]).start()

        merge(1, with_match=False)
        for r in inflight:
            r.wait()
        make_match(rbuf.at[1]).wait()
        make_match_l(rl.at[1]).wait()

        inflight = start_ring(2)
        merge(0, with_match=True)
        for r in inflight:
            r.wait()
        for b in range(B):
            for g in range(Hkv):
                l_run[b][g] = (l_run[b][g]
                               + rl[1, b, g, :, :].astype(jnp.float32)
                               + ll[1, b, g, :, :].astype(jnp.float32))

        wo16 = wo_ref[:].astype(jnp.bfloat16)
        for b in range(B):
            total = None
            for g in range(Hkv):
                inv_l = 1.0 / l_run[b][g]
                for qi in range(G):
                    h_idx = g * G + qi
                    sl = slice(qi * Sq, (qi + 1) * Sq)
                    acc_full = (acc_run_ref[b, g, :, sl]
                                + rbuf[1, b, g, :, sl].astype(jnp.float32)
                                + lbuf[1, b, g, :, sl].astype(jnp.float32))
                    oT = (acc_full * inv_l[:, sl]).astype(jnp.bfloat16)
                    contrib = lax.dot_general(
                        oT, wo16[h_idx * Dh:(h_idx + 1) * Dh, :],
                        (((0,), (0,)), ((), ())),
                        preferred_element_type=jnp.float32)
                    total = contrib if total is None else total + contrib
            out_ref[b, :, :] = total

    return pl.pallas_call(
        body,
        out_shape=jax.ShapeDtypeStruct((B, Sq, Dm), jnp.float32),
        in_specs=[pl.BlockSpec(memory_space=pltpu.VMEM)] * 5,
        out_specs=pl.BlockSpec(memory_space=pltpu.VMEM),
        scratch_shapes=[
            pltpu.VMEM((2, B, Hkv, Dh, R), jnp.bfloat16),
            pltpu.VMEM((2, B, Hkv, Dh, R), jnp.bfloat16),
            pltpu.VMEM((B, Hkv, Dh, R), jnp.bfloat16),
            pltpu.VMEM((2, B, Hkv, 1, R), jnp.bfloat16),
            pltpu.VMEM((2, B, Hkv, 1, R), jnp.bfloat16),
            pltpu.VMEM((B, Hkv, 1, R), jnp.bfloat16),
            pltpu.VMEM((B, Hkv, Dh, R), jnp.float32),
            pltpu.SemaphoreType.DMA((2,)),
            pltpu.SemaphoreType.DMA((2,)),
            pltpu.SemaphoreType.DMA((2,)),
            pltpu.SemaphoreType.DMA((2,)),
            pltpu.SemaphoreType.DMA((2,)),
            pltpu.SemaphoreType.DMA((2,)),
            pltpu.SemaphoreType.DMA((2,)),
            pltpu.SemaphoreType.DMA((2,)),
            pltpu.SemaphoreType.DMA,
            pltpu.SemaphoreType.DMA,
            pltpu.SemaphoreType.DMA,
            pltpu.SemaphoreType.DMA,
        ],
        compiler_params=pltpu.CompilerParams(
            collective_id=0, vmem_limit_bytes=100 * 1024 * 1024),
    )(x, Wq, Wo, K_ext, V_ext)


# device time: 109451 ns/iter; 1.0488x vs baseline; 1.0374x over previous
import jax
import jax.numpy as jnp
from jax import lax
from jax.experimental import pallas as pl
from jax.experimental.pallas import tpu as pltpu

N_DEV = 8
B = 4
Sq = 256
Hq = 8
Hkv = 2
Dh = 128
Dm = 1024
C = 1024
G = Hq // Hkv
R = G * Sq
SCALE = 0.08838834764831843
HOPS = 3

_MESH = pl.DeviceIdType.MESH


def _gray(v):
    return jnp.where(v < 4, v, 11 - v)


def kernel(x, Wq, Wo, K_ext, V_ext):
    def body(x_ref, wq_ref, wo_ref, k_ref, v_ref, out_ref,
             rbuf, lbuf, mbuf, rl, ll, mlb, acc_run_ref,
             r_send, r_recv, l_send, l_recv,
             rl_send, rl_recv, ll_send, ll_recv,
             m_send, m_recv, ml2_send, ml2_recv):
        my = lax.axis_index("i")
        vi = _gray(my)
        right = _gray(lax.rem(vi + 1, N_DEV))
        left = _gray(lax.rem(vi + N_DEV - 1, N_DEV))
        is_even = lax.rem(vi, 2) == 0
        partner_v = lax.rem(vi + jnp.where(is_even, 3, 5), N_DEV)
        partner = _gray(partner_v)

        barrier = pltpu.get_barrier_semaphore()
        for nbr in (left, right, partner):
            pl.semaphore_signal(barrier, inc=1, device_id=(nbr,),
                                device_id_type=_MESH)
        pl.semaphore_wait(barrier, 3)

        xqT = lax.dot_general(
            wq_ref[:], x_ref[:].reshape(B * Sq, Dm),
            (((0,), (1,)), ((), ())),
            preferred_element_type=jnp.float32) * SCALE

        l_run = [[None] * Hkv for _ in range(B)]
        for b in range(B):
            for g in range(Hkv):
                qT = jnp.concatenate(
                    [xqT[(g * G + qi) * Dh:(g * G + qi + 1) * Dh,
                         b * Sq:(b + 1) * Sq]
                     for qi in range(G)], axis=1)
                kg = k_ref[b, :, g, :]
                sT = lax.dot_general(
                    kg, qT, (((1,), (0,)), ((), ())),
                    preferred_element_type=jnp.float32)
                pT = jnp.exp(sT)
                l_b = jnp.sum(pT, axis=0, keepdims=True)
                vg = v_ref[b, :, g, :]
                accT = lax.dot_general(
                    vg, pT, (((0,), (0,)), ((), ())),
                    preferred_element_type=jnp.float32)
                acc_run_ref[b, g, :, :] = accT
                acc16 = accT.astype(jnp.bfloat16)
                l16 = l_b.astype(jnp.bfloat16)
                rbuf[0, b, g, :, :] = acc16
                lbuf[0, b, g, :, :] = acc16
                rl[0, b, g, :, :] = l16
                ll[0, b, g, :, :] = l16
                l_run[b][g] = l_b

        def make(buf, sems_s, sems_r, ss, rs, dst):
            return pltpu.make_async_remote_copy(
                src_ref=buf.at[ss], dst_ref=buf.at[rs],
                send_sem=sems_s.at[ss], recv_sem=sems_r.at[rs],
                device_id=(dst,), device_id_type=_MESH)

        def make_match(src):
            return pltpu.make_async_remote_copy(
                src_ref=src, dst_ref=mbuf,
                send_sem=m_send, recv_sem=m_recv,
                device_id=(partner,), device_id_type=_MESH)

        def make_match_l(src):
            return pltpu.make_async_remote_copy(
                src_ref=src, dst_ref=mlb,
                send_sem=ml2_send, recv_sem=ml2_recv,
                device_id=(partner,), device_id_type=_MESH)

        def start_ring(h):
            ss, rs = h % 2, (h + 1) % 2
            rdmas = [make(rbuf, r_send, r_recv, ss, rs, right),
                     make(rl, rl_send, rl_recv, ss, rs, right),
                     make(lbuf, l_send, l_recv, ss, rs, left),
                     make(ll, ll_send, ll_recv, ss, rs, left)]
            for r in rdmas:
                r.start()
            return rdmas

        def merge(rs, with_match):
            for b in range(B):
                for g in range(Hkv):
                    inc = (rbuf[rs, b, g, :, :].astype(jnp.float32)
                           + lbuf[rs, b, g, :, :].astype(jnp.float32))
                    l_inc = (rl[rs, b, g, :, :].astype(jnp.float32)
                             + ll[rs, b, g, :, :].astype(jnp.float32))
                    if with_match:
                        inc = inc + mbuf[b, g, :, :].astype(jnp.float32)
                        l_inc = l_inc + mlb[b, g, :, :].astype(jnp.float32)
                    acc_run_ref[b, g, :, :] = acc_run_ref[b, g, :, :] + inc
                    l_run[b][g] = l_run[b][g] + l_inc

        inflight = start_ring(0)
        for r in inflight:
            r.wait()

        inflight = start_ring(1)

        @pl.when(is_even)
        def _():
            make_match(rbuf.at[1]).start()
            make_match_l(rl.at[1]).start()

        @pl.when(jnp.logical_not(is_even))
        def _():
            make_match(lbuf.at[1]).start()
            make_match_l(ll.at[1]).start()

        merge(1, with_match=False)
        for r in inflight:
            r.wait()
        make_match(rbuf.at[1]).wait()
        make_match_l(rl.at[1]).wait()

        inflight = start_ring(2)
        merge(0, with_match=True)
        for r in inflight:
            r.wait()
        merge(1, with_match=False)

        for b in range(B):
            total = None
            for g in range(Hkv):
                inv_l = 1.0 / l_run[b][g]
                for qi in range(G):
                    h_idx = g * G + qi
                    oT = (acc_run_ref[b, g, :, qi * Sq:(qi + 1) * Sq]
                          * inv_l[:, qi * Sq:(qi + 1) * Sq])
                    contrib = lax.dot_general(
                        oT, wo_ref[h_idx * Dh:(h_idx + 1) * Dh, :],
                        (((0,), (0,)), ((), ())),
                        preferred_element_type=jnp.float32)
                    total = contrib if total is None else total + contrib
            out_ref[b, :, :] = total

    return pl.pallas_call(
        body,
        out_shape=jax.ShapeDtypeStruct((B, Sq, Dm), jnp.float32),
        in_specs=[pl.BlockSpec(memory_space=pltpu.VMEM)] * 5,
        out_specs=pl.BlockSpec(memory_space=pltpu.VMEM),
        scratch_shapes=[
            pltpu.VMEM((2, B, Hkv, Dh, R), jnp.bfloat16),
            pltpu.VMEM((2, B, Hkv, Dh, R), jnp.bfloat16),
            pltpu.VMEM((B, Hkv, Dh, R), jnp.bfloat16),
            pltpu.VMEM((2, B, Hkv, 1, R), jnp.bfloat16),
            pltpu.VMEM((2, B, Hkv, 1, R), jnp.bfloat16),
            pltpu.VMEM((B, Hkv, 1, R), jnp.bfloat16),
            pltpu.VMEM((B, Hkv, Dh, R), jnp.float32),
            pltpu.SemaphoreType.DMA((2,)),
            pltpu.SemaphoreType.DMA((2,)),
            pltpu.SemaphoreType.DMA((2,)),
            pltpu.SemaphoreType.DMA((2,)),
            pltpu.SemaphoreType.DMA((2,)),
            pltpu.SemaphoreType.DMA((2,)),
            pltpu.SemaphoreType.DMA((2,)),
            pltpu.SemaphoreType.DMA((2,)),
            pltpu.SemaphoreType.DMA,
            pltpu.SemaphoreType.DMA,
            pltpu.SemaphoreType.DMA,
            pltpu.SemaphoreType.DMA,
        ],
        compiler_params=pltpu.CompilerParams(
            collective_id=0, vmem_limit_bytes=100 * 1024 * 1024),
    )(x, Wq, Wo, K_ext, V_ext)
